# baseline (speedup 1.0000x reference)
import functools

import jax
import jax.numpy as jnp
from jax import lax
from jax.experimental import pallas as pl
from jax.experimental.pallas import tpu as pltpu
from jax.experimental.pallas import tpu_sc as plsc

_GA = 128
_SB = 64
_R = 4


@functools.cache
def _build(b, k, d, n_table_rows):
    mesh = plsc.VectorSubcoreMesh(core_axis_name="c", subcore_axis_name="s")
    n_workers = 32
    gb = k - _GA
    rows_per_w = b // n_workers
    n_blocks = rows_per_w // _SB

    @functools.partial(
        pl.kernel,
        out_type=jax.ShapeDtypeStruct((b, k, d), jnp.float32),
        mesh=mesh,
        scratch_types=[
            pltpu.VMEM((_SB, k), jnp.int32),
            pltpu.VMEM((_R, k, d), jnp.float32),
            pltpu.SemaphoreType.DMA,
            pltpu.SemaphoreType.DMA,
            pltpu.SemaphoreType.DMA,
        ],
    )
    def body(table_hbm, idx_hbm, out_hbm, idx_v, rs_v,
             isem, gsem, osem):
        wid = lax.axis_index("s") * 2 + lax.axis_index("c")
        wrow = wid * rows_per_w

        def do_block(ib, _):
            row0 = wrow + ib * _SB
            cp = pltpu.make_async_copy(
                idx_hbm.at[pl.ds(row0, _SB)], idx_v, isem)
            cp.start()
            cp.wait()

            def do_group(g, _):
                j0 = g * _R
                for r in range(_R):
                    pltpu.make_async_copy(
                        table_hbm.at[idx_v.at[j0 + r, pl.ds(0, _GA)]],
                        rs_v.at[r, pl.ds(0, _GA)], gsem).start()
                    pltpu.make_async_copy(
                        table_hbm.at[idx_v.at[j0 + r, pl.ds(_GA, gb)]],
                        rs_v.at[r, pl.ds(_GA, gb)], gsem).start()
                for r in range(_R):
                    pltpu.make_async_copy(
                        table_hbm.at[idx_v.at[j0 + r, pl.ds(0, _GA)]],
                        rs_v.at[r, pl.ds(0, _GA)], gsem).wait()
                    pltpu.make_async_copy(
                        table_hbm.at[idx_v.at[j0 + r, pl.ds(_GA, gb)]],
                        rs_v.at[r, pl.ds(_GA, gb)], gsem).wait()
                    pltpu.make_async_copy(
                        rs_v.at[r],
                        out_hbm.at[row0 + j0 + r],
                        osem).start()
                for r in range(_R):
                    pltpu.make_async_copy(
                        rs_v.at[r],
                        out_hbm.at[row0 + j0 + r],
                        osem).wait()
                return ()

            lax.fori_loop(0, _SB // _R, do_group, (), unroll=False)
            return ()

        lax.fori_loop(0, n_blocks, do_block, (), unroll=False)

    return body


def kernel(idx, id_table):
    b, k = idx.shape
    n_table_rows, d = id_table.shape
    return _build(b, k, d, n_table_rows)(id_table, idx.astype(jnp.int32))


# speedup vs baseline: 1.0047x; 1.0047x over previous
import functools

import jax
import jax.numpy as jnp
from jax import lax
from jax.experimental import pallas as pl
from jax.experimental.pallas import tpu as pltpu
from jax.experimental.pallas import tpu_sc as plsc

_GA = 128
_SB = 32
_R = 4


@functools.cache
def _build(b, k, d, n_table_rows):
    mesh = plsc.VectorSubcoreMesh(core_axis_name="c", subcore_axis_name="s")
    n_workers = 32
    gb = k - _GA
    rows_per_w = b // n_workers
    n_blocks = rows_per_w // _SB

    @functools.partial(
        pl.kernel,
        out_type=jax.ShapeDtypeStruct((b, k, d), jnp.float32),
        mesh=mesh,
        scratch_types=[
            pltpu.VMEM((2, _SB, k), jnp.int32),
            pltpu.VMEM((_R, k, d), jnp.float32),
            pltpu.SemaphoreType.DMA,
            pltpu.SemaphoreType.DMA,
            pltpu.SemaphoreType.DMA,
        ],
    )
    def body(table_hbm, idx_hbm, out_hbm, idx_v, rs_v,
             isem, gsem, osem):
        wid = lax.axis_index("s") * 2 + lax.axis_index("c")
        wrow = wid * rows_per_w

        pltpu.make_async_copy(
            idx_hbm.at[pl.ds(wrow, _SB)], idx_v.at[0], isem).start()

        def do_block(ib, _):
            row0 = wrow + ib * _SB
            cur = lax.rem(ib, 2)
            pltpu.make_async_copy(
                idx_hbm.at[pl.ds(row0, _SB)], idx_v.at[cur], isem).wait()

            @pl.when(ib + 1 < n_blocks)
            def _prefetch():
                pltpu.make_async_copy(
                    idx_hbm.at[pl.ds(row0 + _SB, _SB)],
                    idx_v.at[1 - cur], isem).start()

            def do_group(g, _):
                j0 = g * _R
                for r in range(_R):
                    pltpu.make_async_copy(
                        table_hbm.at[idx_v.at[cur, j0 + r, pl.ds(0, _GA)]],
                        rs_v.at[r, pl.ds(0, _GA)], gsem).start()
                    pltpu.make_async_copy(
                        table_hbm.at[idx_v.at[cur, j0 + r, pl.ds(_GA, gb)]],
                        rs_v.at[r, pl.ds(_GA, gb)], gsem).start()
                for r in range(_R):
                    pltpu.make_async_copy(
                        table_hbm.at[idx_v.at[cur, j0 + r, pl.ds(0, _GA)]],
                        rs_v.at[r, pl.ds(0, _GA)], gsem).wait()
                    pltpu.make_async_copy(
                        table_hbm.at[idx_v.at[cur, j0 + r, pl.ds(_GA, gb)]],
                        rs_v.at[r, pl.ds(_GA, gb)], gsem).wait()
                    pltpu.make_async_copy(
                        rs_v.at[r],
                        out_hbm.at[row0 + j0 + r],
                        osem).start()
                for r in range(_R):
                    pltpu.make_async_copy(
                        rs_v.at[r],
                        out_hbm.at[row0 + j0 + r],
                        osem).wait()
                return ()

            lax.fori_loop(0, _SB // _R, do_group, (), unroll=False)
            return ()

        lax.fori_loop(0, n_blocks, do_block, (), unroll=False)

    return body


def kernel(idx, id_table):
    b, k = idx.shape
    n_table_rows, d = id_table.shape
    return _build(b, k, d, n_table_rows)(id_table, idx.astype(jnp.int32))
